# Initial kernel scaffold; baseline (speedup 1.0000x reference)
import jax
import jax.numpy as jnp
from jax.experimental import pallas as pl


def kernel(x):
    raise NotImplementedError("write your pallas kernel here")


# speedup vs baseline: 15.0193x; 15.0193x over previous
import functools
import jax
import jax.numpy as jnp
from jax.experimental import pallas as pl

_K = 64
_ROWS_PER_BLOCK = 32


def _topk_block(x_ref, o_ref):
    x = x_ref[...]

    def body(i, t_bits):
        bit = 30 - i
        cand = t_bits | (jnp.int32(1) << bit)
        cand_f = jax.lax.bitcast_convert_type(cand, jnp.float32)
        cnt = jnp.sum((x >= cand_f).astype(jnp.int32), axis=1, keepdims=True)
        return jnp.where(cnt >= _K, cand, t_bits)

    t0 = jnp.zeros((x.shape[0], 1), jnp.int32)
    t_bits = jax.lax.fori_loop(0, 31, body, t0, unroll=True)
    t_f = jax.lax.bitcast_convert_type(t_bits, jnp.float32)
    keep = (x >= t_f) & (x > 0.0)
    o_ref[...] = jnp.where(keep, x, 0.0)


@jax.jit
def kernel(x):
    b, n = x.shape
    grid = (b // _ROWS_PER_BLOCK,)
    spec = pl.BlockSpec((_ROWS_PER_BLOCK, n), lambda i: (i, 0))
    return pl.pallas_call(
        _topk_block,
        grid=grid,
        in_specs=[spec],
        out_specs=spec,
        out_shape=jax.ShapeDtypeStruct((b, n), x.dtype),
    )(x)
